# baseline (speedup 1.0000x reference)
import functools

import jax
import jax.numpy as jnp
from jax import lax
from jax.experimental import pallas as pl
from jax.experimental.pallas import tpu as pltpu
from jax.experimental.pallas import tpu_sc as plsc

ROWS = 128
COLS = 32768
L = 16
NW = 32
ROWS_PER_W = ROWS // NW
CHUNKS = COLS // L
U = 8
OUTER = CHUNKS // U


def _argmin_body(x_hbm, out_hbm, row_v, res_v, sem0, sem1):
    c = lax.axis_index("c")
    s = lax.axis_index("s")
    wid = s * 2 + c
    row0 = wid * ROWS_PER_W

    lane = lax.iota(jnp.int32, L)
    sems = (sem0, sem1)

    def start(j):
        pltpu.async_copy(x_hbm.at[row0 + j], row_v.at[j % 2], sems[j % 2])

    start(0)

    for j in range(ROWS_PER_W):
        buf = j % 2
        pltpu.make_async_copy(x_hbm.at[row0 + j], row_v.at[buf], sems[buf]).wait()
        if j + 1 < ROWS_PER_W:
            start(j + 1)

        mvs0 = (jnp.full((L,), jnp.inf, jnp.float32),) * U
        mts0 = (jnp.zeros((L,), jnp.int32),) * U

        @plsc.parallel_loop(0, OUTER, unroll=2, carry=(mvs0, mts0))
        def body(t, carry):
            mvs, mts = carry
            tb = jnp.full((L,), t, jnp.int32)
            new_mvs = []
            new_mts = []
            for k in range(U):
                v = row_v[buf, pl.ds(t * (U * L) + k * L, L)]
                pred = v < mvs[k]
                new_mvs.append(jnp.minimum(mvs[k], v))
                new_mts.append(jnp.where(pred, tb, mts[k]))
            return tuple(new_mvs), tuple(new_mts)

        mvs, mts = body

        mv = mvs[0]
        mi = mts[0] * (U * L) + lane
        for k in range(1, U):
            v2 = mvs[k]
            i2 = mts[k] * (U * L) + (k * L) + lane
            pred = (v2 < mv) | ((v2 == mv) & (i2 < mi))
            mv = jnp.where(pred, v2, mv)
            mi = jnp.where(pred, i2, mi)

        mi_s, mv_s = lax.sort((mi, mv), dimension=0, num_keys=1)
        mv_s2, mi_s2 = lax.sort((mv_s, mi_s), dimension=0, num_keys=1)
        res_v[j, :] = mi_s2

    pltpu.sync_copy(res_v, out_hbm.at[wid])


@functools.partial(jax.jit)
def kernel(x):
    mesh = plsc.VectorSubcoreMesh(core_axis_name="c", subcore_axis_name="s")
    out = pl.kernel(
        _argmin_body,
        out_type=jax.ShapeDtypeStruct((NW, ROWS_PER_W, L), jnp.int32),
        mesh=mesh,
        compiler_params=pltpu.CompilerParams(needs_layout_passes=False, skip_device_barrier=True),
        scratch_types=[
            pltpu.VMEM((2, COLS), jnp.float32),
            pltpu.VMEM((ROWS_PER_W, L), jnp.int32),
            pltpu.SemaphoreType.DMA,
            pltpu.SemaphoreType.DMA,
        ],
    )(x)
    return out[:, :, 0].reshape(ROWS)


# speedup vs baseline: 2.9541x; 2.9541x over previous
import functools

import jax
import jax.numpy as jnp
from jax.experimental import pallas as pl

ROWS = 128
COLS = 32768
BR = 64
GRID = ROWS // BR


def _tc_body(x_ref, o_ref):
    x = x_ref[...]
    idx = jnp.argmin(x, axis=1).astype(jnp.int32)
    o_ref[...] = idx.reshape(1, 1, BR)


@functools.partial(jax.jit)
def kernel(x):
    out = pl.pallas_call(
        _tc_body,
        out_shape=jax.ShapeDtypeStruct((GRID, 1, BR), jnp.int32),
        grid=(GRID,),
        in_specs=[pl.BlockSpec((BR, COLS), lambda i: (i, 0))],
        out_specs=pl.BlockSpec((1, 1, BR), lambda i: (i, 0, 0)),
    )(x)
    return out.reshape(ROWS)
